# baseline (device time: 19566 ns/iter reference)
import jax
import jax.numpy as jnp
from jax import lax
from jax.experimental import pallas as pl
from jax.experimental.pallas import tpu as pltpu

NEG_INF = -1e30


def kernel(Q, K, V, bt, lens):
    B, _, H, D = Q.shape
    P, BS, _, _ = K.shape
    NB = bt.shape[1]
    scale = D ** -0.5

    Q2 = Q.reshape(B, H, D)
    lens2 = lens.reshape(B, 1)

    T = P * BS

    def body(q_ref, k_ref, v_ref, bt_ref, lens_ref, out_ref,
             send_ref, recv_ref, send_sem, recv_sem):
        my_x = lax.axis_index("x")
        peer = (1 - my_x, lax.axis_index("y"), lax.axis_index("z"))

        barrier_sem = pltpu.get_barrier_semaphore()
        pl.semaphore_signal(barrier_sem, inc=1, device_id=peer,
                            device_id_type=pl.DeviceIdType.MESH)
        pl.semaphore_wait(barrier_sem, 1)

        q = q_ref[...]
        btl = bt_ref[...] - my_x * P
        j_iota = lax.broadcasted_iota(jnp.int32, (B, NB), 1)
        valid = (j_iota < lens_ref[...]) & (btl >= 0) & (btl < P)

        btl_v = jnp.where(valid, btl, -1)
        p_idx = lax.broadcasted_iota(jnp.int32, (B, P, NB), 1)
        hits = (btl_v[:, None, :] == p_idx).astype(jnp.float32)
        cnt = jnp.sum(hits, axis=2)
        r_i = lax.broadcasted_iota(jnp.int32, (T, P), 0)
        p_col = lax.broadcasted_iota(jnp.int32, (T, P), 1)
        R = (r_i // BS == p_col).astype(jnp.float32)
        cnt_bt = jax.lax.dot_general(
            cnt, R, (((1,), (1,)), ((), ())),
            preferred_element_type=jnp.float32)
        live = cnt_bt > 0.5

        o_parts, m_parts, s_parts = [], [], []
        for h in range(H):
            khf = k_ref[:, h * D:(h + 1) * D]
            vhf = v_ref[:, h * D:(h + 1) * D]
            s_h = jax.lax.dot_general(
                q[:, h, :], khf, (((1,), (1,)), ((), ())),
                preferred_element_type=jnp.float32) * scale
            s_m = jnp.where(live, s_h, NEG_INF)
            m_h = jnp.max(s_m, axis=1, keepdims=True)
            p_h = jnp.exp(s_m - m_h) * cnt_bt
            s_h_sum = jnp.sum(p_h, axis=1, keepdims=True)
            o_h = jax.lax.dot_general(
                p_h, vhf, (((1,), (0,)), ((), ())),
                preferred_element_type=jnp.float32)
            o_parts.append(o_h[:, None, :])
            m_parts.append(m_h)
            s_parts.append(s_h_sum)

        o = jnp.concatenate(o_parts, axis=1)
        m = jnp.concatenate(m_parts, axis=1)
        s_sum = jnp.concatenate(s_parts, axis=1)

        send_ref[...] = jnp.concatenate(
            [o, m[:, :, None], s_sum[:, :, None]], axis=-1)

        rdma = pltpu.make_async_remote_copy(
            src_ref=send_ref, dst_ref=recv_ref,
            send_sem=send_sem, recv_sem=recv_sem,
            device_id=peer, device_id_type=pl.DeviceIdType.MESH)
        rdma.start()
        rdma.wait()

        r = recv_ref[...]
        o_p, m_p, s_p = r[:, :, :D], r[:, :, D], r[:, :, D + 1]
        m_tot = jnp.maximum(m, m_p)
        a = jnp.exp(m - m_tot)
        a_p = jnp.exp(m_p - m_tot)
        num = o * a[:, :, None] + o_p * a_p[:, :, None]
        den = s_sum * a + s_p * a_p
        out_ref[...] = num / den[:, :, None]

    out = pl.pallas_call(
        body,
        out_shape=jax.ShapeDtypeStruct((B, H, D), jnp.float32),
        in_specs=[pl.BlockSpec(memory_space=pltpu.VMEM)] * 5,
        out_specs=pl.BlockSpec(memory_space=pltpu.VMEM),
        scratch_shapes=[
            pltpu.VMEM((B, H, D + 2), jnp.float32),
            pltpu.VMEM((B, H, D + 2), jnp.float32),
            pltpu.SemaphoreType.DMA,
            pltpu.SemaphoreType.DMA,
        ],
        compiler_params=pltpu.CompilerParams(
            collective_id=0, vmem_limit_bytes=96 * 1024 * 1024),
    )(Q2, K.reshape(P * BS, H * D), V.reshape(P * BS, H * D), bt, lens2)
    return out.reshape(B, 1, H, D)
